# Initial kernel scaffold; baseline (speedup 1.0000x reference)
import jax
import jax.numpy as jnp
from jax.experimental import pallas as pl


def kernel(x, edge_index, W1, b1, W2, b2):
    raise NotImplementedError("write your pallas kernel here")


# speedup vs baseline: 9.9480x; 9.9480x over previous
import functools

import jax
import jax.numpy as jnp
from jax import lax
from jax.experimental import pallas as pl
from jax.experimental.pallas import tpu as pltpu
from jax.experimental.pallas import tpu_sc as plsc

N_NODES = 10000
D = 128
NC = 2
NS = 16
NW = NC * NS
CHUNK = 128
N_PAD = 10240
RPT = N_PAD // NS
G = RPT // CHUNK


def _mesh():
    return plsc.VectorSubcoreMesh(
        core_axis_name="c", subcore_axis_name="s", num_cores=NC, num_subcores=NS
    )


def _fill_slice_indices(idxv, s):
    iota16 = lax.iota(jnp.int32, 16)

    def fill(k, _):
        g = k // 8
        t = k % 8
        idxv[g, pl.ds(t * 16, 16)] = s * RPT + g * CHUNK + t * 16 + iota16
        return _

    lax.fori_loop(0, G * 8, fill, None)


def _deg_body(K, dst_hbm, out_hbm, dstv, onev, zv, idxv, buf, deg_sh, sem):
    c = lax.axis_index("c")
    s = lax.axis_index("s")
    wid = c * NS + s

    zero16 = jnp.zeros((16,), jnp.float32)
    onehot = jnp.where(lax.iota(jnp.int32, 16) == 0, 1.0, 0.0).astype(jnp.float32)

    def init_bufs(i, _):
        zv[i] = zero16
        onev[i] = onehot
        return _

    lax.fori_loop(0, CHUNK, init_bufs, None)
    _fill_slice_indices(idxv, s)

    for g in range(G):
        pltpu.sync_copy(zv, deg_sh.at[idxv.at[g]])
    plsc.subcore_barrier()

    pltpu.sync_copy(dst_hbm.at[pl.ds(wid * K, K)], dstv)

    def step(j, _):
        pltpu.sync_copy(onev, deg_sh.at[dstv.at[j]], add=True)
        return _

    lax.fori_loop(0, K, step, None)
    plsc.subcore_barrier()

    for g in range(G):
        pltpu.async_copy(deg_sh.at[idxv.at[g]], buf, sem).wait()
        pltpu.sync_copy(buf, out_hbm.at[c, pl.ds(s * RPT + g * CHUNK, CHUNK)])


def _make_deg_kernel(K):
    return pl.kernel(
        functools.partial(_deg_body, K),
        out_type=jax.ShapeDtypeStruct((NC, N_PAD, 16), jnp.float32),
        mesh=_mesh(),
        scratch_types=[
            pltpu.VMEM((K, CHUNK), jnp.int32),
            pltpu.VMEM((CHUNK, 16), jnp.float32),
            pltpu.VMEM((CHUNK, 16), jnp.float32),
            pltpu.VMEM((G, CHUNK), jnp.int32),
            pltpu.VMEM((CHUNK, 16), jnp.float32),
            pltpu.VMEM_SHARED((N_PAD, 16), jnp.float32),
            pltpu.SemaphoreType.DMA,
        ],
    )


def _agg_body(K, y_hbm, src_hbm, dst_hbm, out_hbm, srcv, dstv, rows, idxv, acc_sh, sem):
    c = lax.axis_index("c")
    s = lax.axis_index("s")
    wid = c * NS + s

    zero16 = jnp.zeros((16,), jnp.float32)

    def zero_rows(k, _):
        rows[k // 8, pl.ds((k % 8) * 16, 16)] = zero16
        return _

    lax.fori_loop(0, CHUNK * 8, zero_rows, None)
    _fill_slice_indices(idxv, s)

    for g in range(G):
        pltpu.sync_copy(rows, acc_sh.at[idxv.at[g]])
    plsc.subcore_barrier()

    pltpu.sync_copy(src_hbm.at[pl.ds(wid * K, K)], srcv)
    pltpu.sync_copy(dst_hbm.at[pl.ds(wid * K, K)], dstv)

    def step(j, _):
        pltpu.async_copy(y_hbm.at[srcv.at[j]], rows, sem).wait()
        pltpu.sync_copy(rows, acc_sh.at[dstv.at[j]], add=True)
        return _

    lax.fori_loop(0, K, step, None)
    plsc.subcore_barrier()

    for g in range(G):
        pltpu.async_copy(acc_sh.at[idxv.at[g]], rows, sem).wait()
        pltpu.sync_copy(rows, out_hbm.at[c, pl.ds(s * RPT + g * CHUNK, CHUNK)])


def _make_agg_kernel(K):
    return pl.kernel(
        functools.partial(_agg_body, K),
        out_type=jax.ShapeDtypeStruct((NC, N_PAD, D), jnp.float32),
        mesh=_mesh(),
        scratch_types=[
            pltpu.VMEM((K, CHUNK), jnp.int32),
            pltpu.VMEM((K, CHUNK), jnp.int32),
            pltpu.VMEM((CHUNK, D), jnp.float32),
            pltpu.VMEM((G, CHUNK), jnp.int32),
            pltpu.VMEM_SHARED((N_PAD, D), jnp.float32),
            pltpu.SemaphoreType.DMA,
        ],
    )


_BLK = 1280
_GRID = N_PAD // _BLK


def _dis_from(deg_ref):
    deg = jnp.sum(deg_ref[0] + deg_ref[1], axis=1) + 1.0
    return lax.rsqrt(deg)


def _y1_body(x_ref, w_ref, deg_ref, y_ref):
    dis = _dis_from(deg_ref)
    xw = jnp.dot(x_ref[...], w_ref[...], preferred_element_type=jnp.float32)
    y_ref[...] = xw * dis[:, None]


def _mid_body(acc_ref, y_ref, deg_ref, w_ref, b_ref, out_ref):
    dis = _dis_from(deg_ref)
    tot = acc_ref[0] + acc_ref[1] + y_ref[...]
    h = jnp.maximum(tot * dis[:, None] + b_ref[...], 0.0)
    out_ref[...] = jnp.dot(h, w_ref[...], preferred_element_type=jnp.float32) * dis[:, None]


def _fin_body(acc_ref, y_ref, deg_ref, b_ref, out_ref):
    dis = _dis_from(deg_ref)
    tot = acc_ref[0] + acc_ref[1] + y_ref[...]
    out_ref[...] = tot * dis[:, None] + b_ref[...]


_deg_spec = pl.BlockSpec((NC, _BLK, 16), lambda i: (0, i, 0))
_acc_spec = pl.BlockSpec((NC, _BLK, D), lambda i: (0, i, 0))
_row_spec = pl.BlockSpec((_BLK, D), lambda i: (i, 0))
_w_spec = pl.BlockSpec((D, D), lambda i: (0, 0))
_b_spec = pl.BlockSpec((1, D), lambda i: (0, 0))


def _tc_call(body, in_specs):
    return pl.pallas_call(
        body,
        grid=(_GRID,),
        in_specs=in_specs,
        out_specs=_row_spec,
        out_shape=jax.ShapeDtypeStruct((N_PAD, D), jnp.float32),
    )


def kernel(x, edge_index, W1, b1, W2, b2):
    E = edge_index.shape[1]
    K = -(-E // (NW * CHUNK))
    K += K % 2
    e_pad = NW * K * CHUNK - E

    src = edge_index[0].astype(jnp.int32)
    dst = edge_index[1].astype(jnp.int32)
    padv = jnp.full((e_pad,), N_NODES, jnp.int32)
    src_p = jnp.concatenate([src, padv]).reshape(NW * K, CHUNK)
    dst_p = jnp.concatenate([dst, padv]).reshape(NW * K, CHUNK)
    x_p = jnp.pad(x, ((0, N_PAD - N_NODES), (0, 0)))
    b1r = b1.reshape(1, D)
    b2r = b2.reshape(1, D)

    deg = _make_deg_kernel(K)(dst_p)

    y1 = _tc_call(_y1_body, [_row_spec, _w_spec, _deg_spec])(x_p, W1, deg)
    acc1 = _make_agg_kernel(K)(y1, src_p, dst_p)
    y2 = _tc_call(_mid_body, [_acc_spec, _row_spec, _deg_spec, _w_spec, _b_spec])(
        acc1, y1, deg, W2, b1r
    )
    acc2 = _make_agg_kernel(K)(y2, src_p, dst_p)
    out = _tc_call(_fin_body, [_acc_spec, _row_spec, _deg_spec, _b_spec])(
        acc2, y2, deg, b2r
    )
    return out[:N_NODES]


# speedup vs baseline: 10.5516x; 1.0607x over previous
import functools

import jax
import jax.numpy as jnp
from jax import lax
from jax.experimental import pallas as pl
from jax.experimental.pallas import tpu as pltpu
from jax.experimental.pallas import tpu_sc as plsc

N_NODES = 10000
D = 128
NC = 2
NS = 16
NW = NC * NS
CHUNK = 128
N_PAD = 10240
RPT = N_PAD // NS
G = RPT // CHUNK


def _mesh():
    return plsc.VectorSubcoreMesh(
        core_axis_name="c", subcore_axis_name="s", num_cores=NC, num_subcores=NS
    )


def _fill_slice_indices(idxv, s):
    iota16 = lax.iota(jnp.int32, 16)

    def fill(k, _):
        g = k // 8
        t = k % 8
        idxv[g, pl.ds(t * 16, 16)] = s * RPT + g * CHUNK + t * 16 + iota16
        return _

    lax.fori_loop(0, G * 8, fill, None)


def _deg_body(K, dst_hbm, out_hbm, dstv, onev, zv, idxv, buf, deg_sh, sem):
    c = lax.axis_index("c")
    s = lax.axis_index("s")
    wid = c * NS + s

    zero16 = jnp.zeros((16,), jnp.float32)
    onehot = jnp.where(lax.iota(jnp.int32, 16) == 0, 1.0, 0.0).astype(jnp.float32)

    def init_bufs(i, _):
        zv[i] = zero16
        onev[i] = onehot
        return _

    lax.fori_loop(0, CHUNK, init_bufs, None)
    _fill_slice_indices(idxv, s)

    for g in range(G):
        pltpu.sync_copy(zv, deg_sh.at[idxv.at[g]])
    plsc.subcore_barrier()

    def step(j, _):
        pltpu.sync_copy(dst_hbm.at[pl.ds(wid * K + j, 1)], dstv)
        pltpu.sync_copy(onev, deg_sh.at[dstv.at[0]], add=True)
        return _

    lax.fori_loop(0, K, step, None)
    plsc.subcore_barrier()

    for g in range(G):
        pltpu.async_copy(deg_sh.at[idxv.at[g]], buf, sem).wait()
        pltpu.sync_copy(buf, out_hbm.at[c, pl.ds(s * RPT + g * CHUNK, CHUNK)])


def _make_deg_kernel(K):
    return pl.kernel(
        functools.partial(_deg_body, K),
        out_type=jax.ShapeDtypeStruct((NC, N_PAD, 16), jnp.float32),
        mesh=_mesh(),
        scratch_types=[
            pltpu.VMEM((1, CHUNK), jnp.int32),
            pltpu.VMEM((CHUNK, 16), jnp.float32),
            pltpu.VMEM((CHUNK, 16), jnp.float32),
            pltpu.VMEM((G, CHUNK), jnp.int32),
            pltpu.VMEM((CHUNK, 16), jnp.float32),
            pltpu.VMEM_SHARED((N_PAD, 16), jnp.float32),
            pltpu.SemaphoreType.DMA,
        ],
    )


def _agg_body(
    K, y_hbm, src_hbm, dst_hbm, out_hbm, srcv, dstb0, dstb1, rows0, rows1, idxv,
    acc_sh, sem0, sem1, semd0, semd1
):
    c = lax.axis_index("c")
    s = lax.axis_index("s")
    wid = c * NS + s

    zero16 = jnp.zeros((16,), jnp.float32)

    def zero_rows(k, _):
        rows0[k // 8, pl.ds((k % 8) * 16, 16)] = zero16
        return _

    lax.fori_loop(0, CHUNK * 8, zero_rows, None)
    _fill_slice_indices(idxv, s)

    for g in range(G):
        pltpu.sync_copy(rows0, acc_sh.at[idxv.at[g]])
    plsc.subcore_barrier()

    pltpu.sync_copy(src_hbm.at[pl.ds(wid * K, K)], srcv)

    pltpu.async_copy(y_hbm.at[srcv.at[0]], rows0, sem0)
    pltpu.async_copy(dst_hbm.at[pl.ds(wid * K, 1)], dstb0, semd0)

    def step(m, _):
        j0 = 2 * m
        j1 = 2 * m + 1
        pltpu.async_copy(dst_hbm.at[pl.ds(wid * K + j1, 1)], dstb1, semd1)
        pltpu.make_async_copy(y_hbm.at[srcv.at[j0]], rows0, sem0).wait()
        pltpu.async_copy(y_hbm.at[srcv.at[j1]], rows1, sem1)
        pltpu.make_async_copy(dst_hbm.at[pl.ds(0, 1)], dstb0, semd0).wait()
        pltpu.sync_copy(rows0, acc_sh.at[dstb0.at[0]], add=True)
        pltpu.make_async_copy(y_hbm.at[srcv.at[j1]], rows1, sem1).wait()

        @pl.when(m + 1 < K // 2)
        def _start_next():
            pltpu.async_copy(y_hbm.at[srcv.at[j0 + 2]], rows0, sem0)
            pltpu.async_copy(
                dst_hbm.at[pl.ds(wid * K + j0 + 2, 1)], dstb0, semd0
            )

        pltpu.make_async_copy(dst_hbm.at[pl.ds(0, 1)], dstb1, semd1).wait()
        pltpu.sync_copy(rows1, acc_sh.at[dstb1.at[0]], add=True)
        return _

    lax.fori_loop(0, K // 2, step, None)
    plsc.subcore_barrier()

    for g in range(G):
        pltpu.async_copy(acc_sh.at[idxv.at[g]], rows0, sem0).wait()
        pltpu.sync_copy(rows0, out_hbm.at[c, pl.ds(s * RPT + g * CHUNK, CHUNK)])


def _make_agg_kernel(K):
    return pl.kernel(
        functools.partial(_agg_body, K),
        out_type=jax.ShapeDtypeStruct((NC, N_PAD, D), jnp.float32),
        mesh=_mesh(),
        scratch_types=[
            pltpu.VMEM((K, CHUNK), jnp.int32),
            pltpu.VMEM((1, CHUNK), jnp.int32),
            pltpu.VMEM((1, CHUNK), jnp.int32),
            pltpu.VMEM((CHUNK, D), jnp.float32),
            pltpu.VMEM((CHUNK, D), jnp.float32),
            pltpu.VMEM((G, CHUNK), jnp.int32),
            pltpu.VMEM_SHARED((N_PAD, D), jnp.float32),
            pltpu.SemaphoreType.DMA,
            pltpu.SemaphoreType.DMA,
            pltpu.SemaphoreType.DMA,
            pltpu.SemaphoreType.DMA,
        ],
    )


_BLK = 1280
_GRID = N_PAD // _BLK


def _dis_from(deg_ref):
    deg = jnp.sum(deg_ref[0] + deg_ref[1], axis=1) + 1.0
    return lax.rsqrt(deg)


def _y1_body(x_ref, w_ref, deg_ref, y_ref):
    dis = _dis_from(deg_ref)
    xw = jnp.dot(x_ref[...], w_ref[...], preferred_element_type=jnp.float32)
    y_ref[...] = xw * dis[:, None]


def _mid_body(acc_ref, y_ref, deg_ref, w_ref, b_ref, out_ref):
    dis = _dis_from(deg_ref)
    tot = acc_ref[0] + acc_ref[1] + y_ref[...]
    h = jnp.maximum(tot * dis[:, None] + b_ref[...], 0.0)
    out_ref[...] = jnp.dot(h, w_ref[...], preferred_element_type=jnp.float32) * dis[:, None]


def _fin_body(acc_ref, y_ref, deg_ref, b_ref, out_ref):
    dis = _dis_from(deg_ref)
    tot = acc_ref[0] + acc_ref[1] + y_ref[...]
    out_ref[...] = tot * dis[:, None] + b_ref[...]


_deg_spec = pl.BlockSpec((NC, _BLK, 16), lambda i: (0, i, 0))
_acc_spec = pl.BlockSpec((NC, _BLK, D), lambda i: (0, i, 0))
_row_spec = pl.BlockSpec((_BLK, D), lambda i: (i, 0))
_w_spec = pl.BlockSpec((D, D), lambda i: (0, 0))
_b_spec = pl.BlockSpec((1, D), lambda i: (0, 0))


def _tc_call(body, in_specs):
    return pl.pallas_call(
        body,
        grid=(_GRID,),
        in_specs=in_specs,
        out_specs=_row_spec,
        out_shape=jax.ShapeDtypeStruct((N_PAD, D), jnp.float32),
    )


def kernel(x, edge_index, W1, b1, W2, b2):
    E = edge_index.shape[1]
    K = -(-E // (NW * CHUNK))
    K += K % 2
    e_pad = NW * K * CHUNK - E

    src = edge_index[0].astype(jnp.int32)
    dst = edge_index[1].astype(jnp.int32)
    padv = jnp.full((e_pad,), N_NODES, jnp.int32)
    src_p = jnp.concatenate([src, padv]).reshape(NW * K, CHUNK)
    dst_p = jnp.concatenate([dst, padv]).reshape(NW * K, CHUNK)
    x_p = jnp.pad(x, ((0, N_PAD - N_NODES), (0, 0)))
    b1r = b1.reshape(1, D)
    b2r = b2.reshape(1, D)

    deg = _make_deg_kernel(K)(dst_p)

    y1 = _tc_call(_y1_body, [_row_spec, _w_spec, _deg_spec])(x_p, W1, deg)
    acc1 = _make_agg_kernel(K)(y1, src_p, dst_p)
    y2 = _tc_call(_mid_body, [_acc_spec, _row_spec, _deg_spec, _w_spec, _b_spec])(
        acc1, y1, deg, W2, b1r
    )
    acc2 = _make_agg_kernel(K)(y2, src_p, dst_p)
    out = _tc_call(_fin_body, [_acc_spec, _row_spec, _deg_spec, _b_spec])(
        acc2, y2, deg, b2r
    )
    return out[:N_NODES]
